# Initial kernel scaffold; baseline (speedup 1.0000x reference)
import jax
import jax.numpy as jnp
from jax.experimental import pallas as pl


def _body(x_ref, o_ref):
    o_ref[...] = jnp.flip(x_ref[...], axis=1)


def kernel(x, index):
    B, D = x.shape
    RB = 512
    z = pl.pallas_call(
        _body,
        out_shape=jax.ShapeDtypeStruct((B, D), x.dtype),
        grid=(B // RB,),
        in_specs=[pl.BlockSpec((RB, D), lambda i: (i, 0))],
        out_specs=pl.BlockSpec((RB, D), lambda i: (i, 0)),
    )(x)
    log_det = jnp.zeros((B,), dtype=x.dtype)
    return (z, log_det)


# speedup vs baseline: 1.3819x; 1.3819x over previous
import jax
import jax.numpy as jnp
from jax.experimental import pallas as pl


def _body(x_ref, o_ref):
    r = jax.lax.broadcasted_iota(jnp.int32, (128, 128), 0)
    c = jax.lax.broadcasted_iota(jnp.int32, (128, 128), 1)
    flip = (r + c == 127).astype(jnp.float32)
    o_ref[...] = jax.lax.dot(x_ref[...], flip,
                             preferred_element_type=jnp.float32)


def kernel(x, index):
    B, D = x.shape
    RB = 512
    CB = 128
    ncb = D // CB
    z = pl.pallas_call(
        _body,
        out_shape=jax.ShapeDtypeStruct((B, D), x.dtype),
        grid=(B // RB, ncb),
        in_specs=[pl.BlockSpec((RB, CB), lambda i, j: (i, ncb - 1 - j))],
        out_specs=pl.BlockSpec((RB, CB), lambda i, j: (i, j)),
    )(x)
    log_det = jnp.zeros((B,), dtype=x.dtype)
    return (z, log_det)
